# baseline (speedup 1.0000x reference)
import jax
import jax.numpy as jnp
from jax import lax
from jax.experimental import pallas as pl
from jax.experimental.pallas import tpu as pltpu
from jax.experimental.pallas import tpu_sc as plsc

CH = 192
G = 4
P = 9
CG = CH // G
HALF = CG // 2
H = W = 56
Q = H * W
HP = WP = 58
NPIX = HP * WP
T = P * 4
PB = 512
NB = 7
QP = NB * PB
NBATCH = 8
SLABSZ = NPIX * HALF
IWBLK = T * PB
OUTBLK = PB * HALF

_CORNERS = ((0, 0), (1, 0), (0, 1), (1, 1))


def _stage_a_body(x_ref, woff_ref, boff_ref, wmask_ref, bmask_ref,
                  idx_ref, wts_ref):
    blk = pl.program_id(1)
    x = x_ref[0]
    offT = lax.dot_general(woff_ref[...], x, (((0,), (1,)), ((), ())),
                           preferred_element_type=jnp.float32)
    offT = offT + boff_ref[...][:, None]
    mT = lax.dot_general(wmask_ref[...], x, (((0,), (1,)), ((), ())),
                         preferred_element_type=jnp.float32)
    mT = mT + bmask_ref[...][:, None]
    mg = mT.reshape(G, P, PB)
    mg = mg - jnp.max(mg, axis=1, keepdims=True)
    me = jnp.exp(mg)
    m = me / jnp.sum(me, axis=1, keepdims=True)

    q = blk * PB + lax.broadcasted_iota(jnp.int32, (1, PB), 1)
    fi = (q // W).astype(jnp.float32)
    fj = (q % W).astype(jnp.float32)
    lin = (-1.0, 0.0, 1.0)

    for g in range(G):
        for p in range(P):
            gp = g * P + p
            px = offT[2 * gp:2 * gp + 1, :]
            py = offT[2 * gp + 1:2 * gp + 2, :]
            ix = fj + (1.0 + lin[p // 3]) + px
            iy = fi + (1.0 + lin[p % 3]) + py
            ix0 = jnp.floor(ix)
            iy0 = jnp.floor(iy)
            wx1 = ix - ix0
            wx0 = 1.0 - wx1
            wy1 = iy - iy0
            wy0 = 1.0 - wy1
            mgp = m[g, p].reshape(1, PB)
            for k, (dx, dy) in enumerate(_CORNERS):
                ixk = ix0 + dx
                iyk = iy0 + dy
                valid = ((ixk >= 0.0) & (ixk <= WP - 1.0)
                         & (iyk >= 0.0) & (iyk <= HP - 1.0))
                ixc = jnp.clip(ixk, 0.0, WP - 1.0)
                iyc = jnp.clip(iyk, 0.0, HP - 1.0)
                row = iyc * float(WP) + ixc
                wxy = (wx1 if dx else wx0) * (wy1 if dy else wy0)
                wk = wxy * mgp * valid.astype(jnp.float32)
                t = p * 4 + k
                off = (g * T + t) * PB
                idx_ref[pl.ds(off, PB)] = (
                    (row * float(HALF)).astype(jnp.int32).reshape(PB))
                wts_ref[pl.ds(off, PB)] = wk.reshape(PB)


def _stage_a(x2p, W_off, b_off, W_mask, b_mask):
    return pl.pallas_call(
        _stage_a_body,
        grid=(NBATCH, NB),
        in_specs=[
            pl.BlockSpec((1, PB, CH), lambda n, b: (n, b, 0)),
            pl.BlockSpec((CH, G * P * 2), lambda n, b: (0, 0)),
            pl.BlockSpec((G * P * 2,), lambda n, b: (0,)),
            pl.BlockSpec((CH, G * P), lambda n, b: (0, 0)),
            pl.BlockSpec((G * P,), lambda n, b: (0,)),
        ],
        out_specs=[
            pl.BlockSpec((G * IWBLK,), lambda n, b: (n * NB + b,)),
            pl.BlockSpec((G * IWBLK,), lambda n, b: (n * NB + b,)),
        ],
        out_shape=[
            jax.ShapeDtypeStruct((NBATCH * NB * G * IWBLK,), jnp.int32),
            jax.ShapeDtypeStruct((NBATCH * NB * G * IWBLK,), jnp.float32),
        ],
    )(x2p, W_off, b_off, W_mask, b_mask)


def _stage_b_body(slabs_hbm, idx_hbm, wts_hbm, out_hbm,
                  slab_v, idx_v, wts_v, out_v):
    wid = lax.axis_index("c") * 16 + lax.axis_index("s")
    lane24 = lax.iota(jnp.int32, 16) * HALF

    for s in range(2):
        slab_id = wid * 2 + s
        n = slab_id // (G * 2)
        gh = slab_id % (G * 2)
        g = gh // 2
        pltpu.sync_copy(slabs_hbm.at[pl.ds(slab_id * SLABSZ, SLABSZ)],
                        slab_v)

        def block_body(blk, _):
            iw_off = ((n * NB + blk) * G + g) * IWBLK
            pltpu.sync_copy(idx_hbm.at[pl.ds(iw_off, IWBLK)], idx_v)
            pltpu.sync_copy(wts_hbm.at[pl.ds(iw_off, IWBLK)], wts_v)

            def pg_body(pg, _):
                iv0 = pg * 16

                def tap_body(t, acc):
                    iv = idx_v[pl.ds(t * PB + iv0, 16)]
                    wv = wts_v[pl.ds(t * PB + iv0, 16)]
                    return tuple(
                        acc[c] + wv * plsc.load_gather(slab_v, [iv + c])
                        for c in range(HALF))

                z = jnp.zeros((16,), jnp.float32)
                acc = lax.fori_loop(0, T, tap_body, (z,) * HALF)
                obase = lane24 + iv0 * HALF
                for c in range(HALF):
                    plsc.store_scatter(out_v, [obase + c], acc[c])
                return 0

            npg = lax.select(blk < NB - 1, PB // 16, (Q - (NB - 1) * PB) // 16)
            lax.fori_loop(0, npg, pg_body, 0)
            pltpu.sync_copy(
                out_v,
                out_hbm.at[pl.ds(((n * (G * 2) + gh) * NB + blk) * OUTBLK,
                                 OUTBLK)])
            return 0

        lax.fori_loop(0, NB, block_body, 0)


def _stage_b(slabs, idx, wts):
    mesh = plsc.VectorSubcoreMesh(core_axis_name="c", subcore_axis_name="s")
    return pl.kernel(
        _stage_b_body,
        out_type=jax.ShapeDtypeStruct((NBATCH * G * 2 * NB * OUTBLK,),
                                      jnp.float32),
        mesh=mesh,
        compiler_params=pltpu.CompilerParams(
            use_tc_tiling_on_sc=False, needs_layout_passes=False),
        scratch_types=[
            pltpu.VMEM((SLABSZ,), jnp.float32),
            pltpu.VMEM((IWBLK,), jnp.int32),
            pltpu.VMEM((IWBLK,), jnp.float32),
            pltpu.VMEM((OUTBLK,), jnp.float32),
        ],
    )(slabs, idx, wts)


def kernel(input, W_off, b_off, W_mask, b_mask):
    b = input.shape[0]
    x2 = input.reshape(b, Q, CH)
    x2p = jnp.pad(x2, ((0, 0), (0, QP - Q), (0, 0)))
    idx, wts = _stage_a(x2p, W_off, b_off, W_mask, b_mask)
    xpad = jnp.pad(x2.reshape(b, H, W, CH),
                   ((0, 0), (1, 1), (1, 1), (0, 0)))
    slabs = (xpad.reshape(b, NPIX, G * 2, HALF)
             .transpose(0, 2, 1, 3)
             .reshape(b * G * 2 * SLABSZ))
    out = _stage_b(slabs, idx, wts)
    out = (out.reshape(b, G * 2, QP, HALF)[:, :, :Q, :]
           .transpose(0, 2, 1, 3)
           .reshape(b, Q, CH))
    return out.reshape(b, CH, H, W)


# speedup vs baseline: 1.1924x; 1.1924x over previous
import jax
import jax.numpy as jnp
from jax import lax
from jax.experimental import pallas as pl
from jax.experimental.pallas import tpu as pltpu
from jax.experimental.pallas import tpu_sc as plsc

CH = 192
G = 4
P = 9
CG = CH // G
HALF = CG // 2
H = W = 56
Q = H * W
HP = WP = 58
NPIX = HP * WP
T = P * 4
PB = 448
NB = Q // PB
NBATCH = 8

_CORNERS = ((0, 0), (1, 0), (0, 1), (1, 1))


def _stage_a_body(x_ref, woff_ref, boff_ref, wmask_ref, bmask_ref,
                  idx_ref, wts_ref):
    blk = pl.program_id(1)
    x = x_ref[0]
    offT = lax.dot_general(woff_ref[...], x, (((0,), (1,)), ((), ())),
                           preferred_element_type=jnp.float32)
    offT = offT + boff_ref[...][:, None]
    mT = lax.dot_general(wmask_ref[...], x, (((0,), (1,)), ((), ())),
                         preferred_element_type=jnp.float32)
    mT = mT + bmask_ref[...][:, None]
    mg = mT.reshape(G, P, PB)
    mg = mg - jnp.max(mg, axis=1, keepdims=True)
    me = jnp.exp(mg)
    m = me / jnp.sum(me, axis=1, keepdims=True)

    q = blk * PB + lax.broadcasted_iota(jnp.int32, (1, PB), 1)
    fi = (q // W).astype(jnp.float32)
    fj = (q % W).astype(jnp.float32)
    lin = (-1.0, 0.0, 1.0)

    for g in range(G):
        idx_rows = []
        wts_rows = []
        for p in range(P):
            gp = g * P + p
            px = offT[2 * gp:2 * gp + 1, :]
            py = offT[2 * gp + 1:2 * gp + 2, :]
            ix = fj + (1.0 + lin[p // 3]) + px
            iy = fi + (1.0 + lin[p % 3]) + py
            ix0 = jnp.floor(ix)
            iy0 = jnp.floor(iy)
            wx1 = ix - ix0
            wx0 = 1.0 - wx1
            wy1 = iy - iy0
            wy0 = 1.0 - wy1
            mgp = m[g, p].reshape(1, PB)
            for dx, dy in _CORNERS:
                ixk = ix0 + dx
                iyk = iy0 + dy
                valid = ((ixk >= 0.0) & (ixk <= WP - 1.0)
                         & (iyk >= 0.0) & (iyk <= HP - 1.0))
                ixc = jnp.clip(ixk, 0.0, WP - 1.0)
                iyc = jnp.clip(iyk, 0.0, HP - 1.0)
                row = iyc * float(WP) + ixc
                wxy = (wx1 if dx else wx0) * (wy1 if dy else wy0)
                wk = wxy * mgp * valid.astype(jnp.float32)
                idx_rows.append((row * float(HALF)).astype(jnp.int32))
                wts_rows.append(wk)
        idx_ref[0, 0, g] = jnp.concatenate(idx_rows, axis=0)
        wts_ref[0, 0, g] = jnp.concatenate(wts_rows, axis=0)


def _stage_a(x2, W_off, b_off, W_mask, b_mask):
    return pl.pallas_call(
        _stage_a_body,
        grid=(NBATCH, NB),
        in_specs=[
            pl.BlockSpec((1, PB, CH), lambda n, b: (n, b, 0)),
            pl.BlockSpec((CH, G * P * 2), lambda n, b: (0, 0)),
            pl.BlockSpec((G * P * 2,), lambda n, b: (0,)),
            pl.BlockSpec((CH, G * P), lambda n, b: (0, 0)),
            pl.BlockSpec((G * P,), lambda n, b: (0,)),
        ],
        out_specs=[
            pl.BlockSpec((1, 1, G, T, PB), lambda n, b: (n, b, 0, 0, 0)),
            pl.BlockSpec((1, 1, G, T, PB), lambda n, b: (n, b, 0, 0, 0)),
        ],
        out_shape=[
            jax.ShapeDtypeStruct((NBATCH, NB, G, T, PB), jnp.int32),
            jax.ShapeDtypeStruct((NBATCH, NB, G, T, PB), jnp.float32),
        ],
    )(x2, W_off, b_off, W_mask, b_mask)


def _stage_b_body(slabs_hbm, idx_hbm, wts_hbm, out_hbm,
                  slab_v, idx_v, wts_v, out_v, sem):
    wid = lax.axis_index("c") * 16 + lax.axis_index("s")
    lane16 = lax.iota(jnp.int32, 16)

    for s in range(2):
        slab_id = wid * 2 + s
        n = slab_id // (G * 2)
        gh = slab_id % (G * 2)
        g = gh // 2
        pltpu.sync_copy(slabs_hbm.at[n, gh], slab_v)

        def block_body(blk, _):
            ci = pltpu.async_copy(idx_hbm.at[n, blk, g], idx_v, sem)
            cw = pltpu.async_copy(wts_hbm.at[n, blk, g], wts_v, sem)
            ci.wait()
            cw.wait()

            def pg_body(pg, _):
                iv0 = pg * 16
                rows = iv0 + lane16

                def tap_body(t, acc):
                    iv = idx_v[t, pl.ds(iv0, 16)]
                    wv = wts_v[t, pl.ds(iv0, 16)]
                    return tuple(
                        acc[c] + wv * plsc.load_gather(slab_v, [iv + c])
                        for c in range(HALF))

                z = jnp.zeros((16,), jnp.float32)
                acc = lax.fori_loop(0, T, tap_body, (z,) * HALF)
                for c in range(HALF):
                    plsc.store_scatter(
                        out_v, [rows, jnp.full((16,), c, jnp.int32)], acc[c])
                return 0

            lax.fori_loop(0, PB // 16, pg_body, 0)
            pltpu.sync_copy(
                out_v,
                out_hbm.at[n, pl.ds(blk * PB, PB), pl.ds(gh * HALF, HALF)])
            return 0

        lax.fori_loop(0, NB, block_body, 0)


def _stage_b(slabs, idx, wts):
    mesh = plsc.VectorSubcoreMesh(core_axis_name="c", subcore_axis_name="s")
    return pl.kernel(
        _stage_b_body,
        out_type=jax.ShapeDtypeStruct((NBATCH, Q, CH), jnp.float32),
        mesh=mesh,
        compiler_params=pltpu.CompilerParams(
            use_tc_tiling_on_sc=False, needs_layout_passes=False),
        scratch_types=[
            pltpu.VMEM((NPIX * HALF,), jnp.float32),
            pltpu.VMEM((T, PB), jnp.int32),
            pltpu.VMEM((T, PB), jnp.float32),
            pltpu.VMEM((PB, HALF), jnp.float32),
            pltpu.SemaphoreType.DMA,
        ],
    )(slabs, idx, wts)


def kernel(input, W_off, b_off, W_mask, b_mask):
    b = input.shape[0]
    x2 = input.reshape(b, Q, CH)
    idx, wts = _stage_a(x2, W_off, b_off, W_mask, b_mask)
    xpad = jnp.pad(x2.reshape(b, H, W, CH),
                   ((0, 0), (1, 1), (1, 1), (0, 0)))
    slabs = (xpad.reshape(b, NPIX, G * 2, HALF)
             .transpose(0, 2, 1, 3)
             .reshape(b, G * 2, NPIX * HALF))
    out = _stage_b(slabs, idx, wts)
    return out.reshape(b, CH, H, W)


# speedup vs baseline: 1.1994x; 1.0059x over previous
import jax
import jax.numpy as jnp
from jax import lax
from jax.experimental import pallas as pl
from jax.experimental.pallas import tpu as pltpu
from jax.experimental.pallas import tpu_sc as plsc

CH = 192
G = 4
P = 9
CG = CH // G
HALF = CG // 2
H = W = 56
Q = H * W
HP = WP = 58
NPIX = HP * WP
T = P * 4
PB = 448
NB = Q // PB
NBATCH = 8

_CORNERS = ((0, 0), (1, 0), (0, 1), (1, 1))


def _stage_a_body(x_ref, woff_ref, boff_ref, wmask_ref, bmask_ref,
                  idx_ref, wts_ref):
    blk = pl.program_id(1)
    x = x_ref[0]
    offT = lax.dot_general(woff_ref[...], x, (((0,), (1,)), ((), ())),
                           preferred_element_type=jnp.float32)
    offT = offT + boff_ref[...][:, None]
    mT = lax.dot_general(wmask_ref[...], x, (((0,), (1,)), ((), ())),
                         preferred_element_type=jnp.float32)
    mT = mT + bmask_ref[...][:, None]
    mg = mT.reshape(G, P, PB)
    mg = mg - jnp.max(mg, axis=1, keepdims=True)
    me = jnp.exp(mg)
    m = me / jnp.sum(me, axis=1, keepdims=True)

    q = blk * PB + lax.broadcasted_iota(jnp.int32, (1, PB), 1)
    fi = (q // W).astype(jnp.float32)
    fj = (q % W).astype(jnp.float32)
    lin = (-1.0, 0.0, 1.0)

    for g in range(G):
        idx_rows = []
        wts_rows = []
        for p in range(P):
            gp = g * P + p
            px = offT[2 * gp:2 * gp + 1, :]
            py = offT[2 * gp + 1:2 * gp + 2, :]
            ix = fj + (1.0 + lin[p // 3]) + px
            iy = fi + (1.0 + lin[p % 3]) + py
            ix0 = jnp.floor(ix)
            iy0 = jnp.floor(iy)
            wx1 = ix - ix0
            wx0 = 1.0 - wx1
            wy1 = iy - iy0
            wy0 = 1.0 - wy1
            mgp = m[g, p].reshape(1, PB)
            for dx, dy in _CORNERS:
                ixk = ix0 + dx
                iyk = iy0 + dy
                valid = ((ixk >= 0.0) & (ixk <= WP - 1.0)
                         & (iyk >= 0.0) & (iyk <= HP - 1.0))
                ixc = jnp.clip(ixk, 0.0, WP - 1.0)
                iyc = jnp.clip(iyk, 0.0, HP - 1.0)
                row = iyc * float(WP) + ixc
                wxy = (wx1 if dx else wx0) * (wy1 if dy else wy0)
                wk = wxy * mgp * valid.astype(jnp.float32)
                idx_rows.append((row * float(HALF)).astype(jnp.int32))
                wts_rows.append(wk)
        idx_ref[0, 0, g] = jnp.concatenate(idx_rows, axis=0)
        wts_ref[0, 0, g] = jnp.concatenate(wts_rows, axis=0)


def _stage_a(x2, W_off, b_off, W_mask, b_mask):
    return pl.pallas_call(
        _stage_a_body,
        grid=(NBATCH, NB),
        in_specs=[
            pl.BlockSpec((1, PB, CH), lambda n, b: (n, b, 0)),
            pl.BlockSpec((CH, G * P * 2), lambda n, b: (0, 0)),
            pl.BlockSpec((G * P * 2,), lambda n, b: (0,)),
            pl.BlockSpec((CH, G * P), lambda n, b: (0, 0)),
            pl.BlockSpec((G * P,), lambda n, b: (0,)),
        ],
        out_specs=[
            pl.BlockSpec((1, 1, G, T, PB), lambda n, b: (n, b, 0, 0, 0)),
            pl.BlockSpec((1, 1, G, T, PB), lambda n, b: (n, b, 0, 0, 0)),
        ],
        out_shape=[
            jax.ShapeDtypeStruct((NBATCH, NB, G, T, PB), jnp.int32),
            jax.ShapeDtypeStruct((NBATCH, NB, G, T, PB), jnp.float32),
        ],
    )(x2, W_off, b_off, W_mask, b_mask)


def _stage_b_body(slabs_hbm, idx_hbm, wts_hbm, out_hbm,
                  slab_v, idx_v, wts_v, out_v, sem, sem_o):
    wid = lax.axis_index("c") * 16 + lax.axis_index("s")
    lane16 = lax.iota(jnp.int32, 16)

    for s in range(2):
        slab_id = wid * 2 + s
        n = slab_id // (G * 2)
        gh = slab_id % (G * 2)
        g = gh // 2
        pltpu.sync_copy(slabs_hbm.at[n, gh], slab_v)

        def out_copy(blk):
            return pltpu.make_async_copy(
                out_v,
                out_hbm.at[n, pl.ds(blk * PB, PB), pl.ds(gh * HALF, HALF)],
                sem_o)

        def block_body(blk, _):
            ci = pltpu.async_copy(idx_hbm.at[n, blk, g], idx_v, sem)
            cw = pltpu.async_copy(wts_hbm.at[n, blk, g], wts_v, sem)

            @pl.when(blk > 0)
            def _():
                out_copy(blk - 1).wait()

            ci.wait()
            cw.wait()

            def pg_body(pg, _):
                iv0 = pg * 16
                rows = iv0 + lane16

                def tap_body(t, acc):
                    iv = idx_v[t, pl.ds(iv0, 16)]
                    wv = wts_v[t, pl.ds(iv0, 16)]
                    return tuple(
                        acc[c] + wv * plsc.load_gather(slab_v, [iv + c])
                        for c in range(HALF))

                z = jnp.zeros((16,), jnp.float32)
                acc = lax.fori_loop(0, T, tap_body, (z,) * HALF)
                for c in range(HALF):
                    plsc.store_scatter(
                        out_v, [rows, jnp.full((16,), c, jnp.int32)], acc[c])
                return 0

            lax.fori_loop(0, PB // 16, pg_body, 0)
            out_copy(blk).start()
            return 0

        lax.fori_loop(0, NB, block_body, 0)
        out_copy(NB - 1).wait()


def _stage_b(slabs, idx, wts):
    mesh = plsc.VectorSubcoreMesh(core_axis_name="c", subcore_axis_name="s")
    return pl.kernel(
        _stage_b_body,
        out_type=jax.ShapeDtypeStruct((NBATCH, Q, CH), jnp.float32),
        mesh=mesh,
        compiler_params=pltpu.CompilerParams(
            use_tc_tiling_on_sc=False, needs_layout_passes=False),
        scratch_types=[
            pltpu.VMEM((NPIX * HALF,), jnp.float32),
            pltpu.VMEM((T, PB), jnp.int32),
            pltpu.VMEM((T, PB), jnp.float32),
            pltpu.VMEM((PB, HALF), jnp.float32),
            pltpu.SemaphoreType.DMA,
            pltpu.SemaphoreType.DMA,
        ],
    )(slabs, idx, wts)


def kernel(input, W_off, b_off, W_mask, b_mask):
    b = input.shape[0]
    x2 = input.reshape(b, Q, CH)
    idx, wts = _stage_a(x2, W_off, b_off, W_mask, b_mask)
    xpad = jnp.pad(x2.reshape(b, H, W, CH),
                   ((0, 0), (1, 1), (1, 1), (0, 0)))
    slabs = (xpad.reshape(b, NPIX, G * 2, HALF)
             .transpose(0, 2, 1, 3)
             .reshape(b, G * 2, NPIX * HALF))
    out = _stage_b(slabs, idx, wts)
    return out.reshape(b, CH, H, W)
